# baseline (speedup 1.0000x reference)
import functools

import jax
import jax.numpy as jnp
from jax import lax
from jax.experimental import pallas as pl
from jax.experimental.pallas import tpu as pltpu
from jax.experimental.pallas import tpu_sc as plsc

B, K, L = 1, 512, 256
VOCAB, C_M, C_Z = 5, 256, 128
D_EMB = C_Z // 2
MAXREL = 32
VPAD = 8
RELPAD = 72
TOK_BLK = 4096
E_ROWS = 512
NW = 32
L_SC = 128
L_PER_W = L_SC // NW
L_BLK = 16


def _msa_body(tok_ref, tab_ref, out_ref):
    t = jnp.broadcast_to(tok_ref[...].reshape(1, TOK_BLK), (VPAD, TOK_BLK))
    vrow = jax.lax.broadcasted_iota(jnp.int32, (VPAD, TOK_BLK), 0)
    onehot_t = (t == vrow).astype(jnp.float32)
    out_ref[...] = jax.lax.dot_general(
        onehot_t, tab_ref[...], (((0,), (0,)), ((), ())),
        preferred_element_type=jnp.float32)


def _pre_body(ptok_ref, ptab_ref, wl_ref, wr_ref, b_ref,
              lp_ref, rpb_ref):
    t = jnp.broadcast_to(ptok_ref[...].reshape(1, L), (VPAD, L))
    vrow = jax.lax.broadcasted_iota(jnp.int32, (VPAD, L), 0)
    onehot_t = (t == vrow).astype(jnp.float32)
    seq_emb = jax.lax.dot_general(
        onehot_t, ptab_ref[...], (((0,), (0,)), ((), ())),
        preferred_element_type=jnp.float32)
    lp_ref[...] = jnp.dot(seq_emb, wl_ref[...],
                          preferred_element_type=jnp.float32)
    rpb_ref[...] = jnp.dot(seq_emb, wr_ref[...],
                           preferred_element_type=jnp.float32) + b_ref[...]


def _sc_pair_body(lp_hbm, rpb_hbm, rel_hbm, out_hbm,
                  lp_v, rpb_v, rel_v, out_v, sem0, sem1, sem_in):
    cid = lax.axis_index("c")
    sid = lax.axis_index("s")
    wid = sid * 2 + cid
    l0 = wid * L_PER_W
    pltpu.async_copy(lp_hbm.at[pl.ds(l0, L_PER_W)], lp_v, sem_in).wait()
    pltpu.async_copy(rpb_hbm.at[pl.ds(0, L)], rpb_v, sem_in).wait()
    pltpu.async_copy(rel_hbm.at[pl.ds(0, RELPAD)], rel_v, sem_in).wait()

    sems = (sem0, sem1)
    pending = [None, None]
    for j in range(L_PER_W):
        p = j % 2
        if pending[p] is not None:
            pending[p].wait()
        l = l0 + j
        lo = lax.max(0, l - MAXREL)
        hi = lax.min(L - 1, l + MAXREL)
        lpc = [lp_v[j, pl.ds(c * 16, 16)] for c in range(8)]
        base0 = [lpc[c] + rel_v[0, pl.ds(c * 16, 16)] for c in range(8)]
        base64 = [lpc[c] + rel_v[2 * MAXREL, pl.ds(c * 16, 16)]
                  for c in range(8)]

        @plsc.parallel_loop(0, lo, unroll=4)
        def _lo_loop(m):
            vals = [rpb_v[m, pl.ds(c * 16, 16)] for c in range(8)]
            for c in range(8):
                out_v[p, m, pl.ds(c * 16, 16)] = vals[c] + base0[c]

        @plsc.parallel_loop(lo, hi + 1, unroll=2)
        def _band_loop(m):
            srow = m - l + MAXREL
            a = [rpb_v[m, pl.ds(c * 16, 16)] for c in range(8)]
            b = [rel_v[srow, pl.ds(c * 16, 16)] for c in range(8)]
            for c in range(8):
                out_v[p, m, pl.ds(c * 16, 16)] = a[c] + b[c] + lpc[c]

        @plsc.parallel_loop(hi + 1, L, unroll=4)
        def _hi_loop(m):
            vals = [rpb_v[m, pl.ds(c * 16, 16)] for c in range(8)]
            for c in range(8):
                out_v[p, m, pl.ds(c * 16, 16)] = vals[c] + base64[c]
        cp = pltpu.async_copy(out_v.at[p],
                              out_hbm.at[pl.ds(l * L, L)], sems[p])
        pending[p] = cp
    pending[0].wait()
    pending[1].wait()


def _tc_pair_body(ptok_ref, ptok_blk_ref, ptab_ref, wl_ref, wr_ref, b_ref,
                  rel_ref, buf_ref, out_ref):
    i = pl.program_id(0)
    t = jnp.broadcast_to(ptok_ref[...].reshape(1, L), (VPAD, L))
    vrow = jax.lax.broadcasted_iota(jnp.int32, (VPAD, L), 0)
    onehot_t = (t == vrow).astype(jnp.float32)
    seq_emb = jax.lax.dot_general(
        onehot_t, ptab_ref[...], (((0,), (0,)), ((), ())),
        preferred_element_type=jnp.float32)
    rpb = jnp.dot(seq_emb, wr_ref[...], preferred_element_type=jnp.float32)
    rpb = rpb + b_ref[...]
    tb = jnp.broadcast_to(ptok_blk_ref[...], (L_BLK, VPAD))
    lane_b = jax.lax.broadcasted_iota(jnp.int32, (L_BLK, VPAD), 1)
    se_blk = jnp.dot((tb == lane_b).astype(jnp.float32), ptab_ref[...],
                     preferred_element_type=jnp.float32)
    lp = jnp.dot(se_blk, wl_ref[...], preferred_element_type=jnp.float32)
    r = jax.lax.broadcasted_iota(jnp.int32, (1, L_BLK * L), 1)
    m = r & (L - 1)
    lrow = (r >> 8) + (L_SC + i * L_BLK)
    s = jnp.clip(m - lrow, -MAXREL, MAXREL) + MAXREL
    sb = jnp.broadcast_to(s, (RELPAD, L_BLK * L))
    vrow2 = jax.lax.broadcasted_iota(jnp.int32, (RELPAD, L_BLK * L), 0)
    ohr_t = (sb == vrow2).astype(jnp.float32)
    rp = jax.lax.dot_general(
        ohr_t, rel_ref[...], (((0,), (0,)), ((), ())),
        preferred_element_type=jnp.float32)
    out = rp.reshape(L_BLK, L, C_Z) + lp[:, None, :] + rpb[None, :, :]
    out_ref[...] = out.reshape(L_BLK * L, C_Z)


_sc_pair = functools.partial(
    pl.kernel,
    out_type=jax.ShapeDtypeStruct((L * L, C_Z), jnp.float32),
    mesh=plsc.VectorSubcoreMesh(core_axis_name="c", subcore_axis_name="s"),
    scratch_types=[
        pltpu.VMEM((L_PER_W, C_Z), jnp.float32),
        pltpu.VMEM((L, C_Z), jnp.float32),
        pltpu.VMEM((RELPAD, C_Z), jnp.float32),
        pltpu.VMEM((2, L, C_Z), jnp.float32),
        pltpu.SemaphoreType.DMA,
        pltpu.SemaphoreType.DMA,
        pltpu.SemaphoreType.DMA,
    ],
)(_sc_pair_body)


@jax.jit
def kernel(tokens, pair_tokens, msa_table, pair_table, proj_W, proj_b,
           relpos_table):
    tok_flat = tokens.reshape(K * L // TOK_BLK, 1, TOK_BLK).astype(jnp.int32)
    msa_pad = jnp.zeros((VPAD, C_M), jnp.float32).at[:VOCAB].set(msa_table)
    ptok_w = pair_tokens.reshape(1, 1, L).astype(jnp.int32)
    ptok_blk = pair_tokens.reshape(L, 1).astype(jnp.int32)
    ptab = jnp.zeros((VPAD, D_EMB), jnp.float32).at[:VOCAB].set(pair_table)
    wl_t = proj_W[:, :D_EMB].T
    wr_t = proj_W[:, D_EMB:].T
    b2 = proj_b.reshape(1, C_Z)
    rel_pad = jnp.zeros((RELPAD, C_Z), jnp.float32).at[:2 * MAXREL + 1].set(
        relpos_table)

    lp, rpb = pl.pallas_call(
        _pre_body,
        grid=(1,),
        in_specs=[
            pl.BlockSpec((1, 1, L), lambda i: (0, 0, 0)),
            pl.BlockSpec((VPAD, D_EMB), lambda i: (0, 0)),
            pl.BlockSpec((D_EMB, C_Z), lambda i: (0, 0)),
            pl.BlockSpec((D_EMB, C_Z), lambda i: (0, 0)),
            pl.BlockSpec((1, C_Z), lambda i: (0, 0)),
        ],
        out_specs=[
            pl.BlockSpec((L, C_Z), lambda i: (0, 0)),
            pl.BlockSpec((L, C_Z), lambda i: (0, 0)),
        ],
        out_shape=[
            jax.ShapeDtypeStruct((L, C_Z), jnp.float32),
            jax.ShapeDtypeStruct((L, C_Z), jnp.float32),
        ],
    )(ptok_w, ptab, wl_t, wr_t, b2)

    pair_low = _sc_pair(lp, rpb, rel_pad)

    pair = pl.pallas_call(
        _tc_pair_body,
        grid=((L - L_SC) // L_BLK,),
        in_specs=[
            pl.BlockSpec((1, 1, L), lambda i: (0, 0, 0)),
            pl.BlockSpec((L_BLK, 1), lambda i: (i + L_SC // L_BLK, 0)),
            pl.BlockSpec((VPAD, D_EMB), lambda i: (0, 0)),
            pl.BlockSpec((D_EMB, C_Z), lambda i: (0, 0)),
            pl.BlockSpec((D_EMB, C_Z), lambda i: (0, 0)),
            pl.BlockSpec((1, C_Z), lambda i: (0, 0)),
            pl.BlockSpec((RELPAD, C_Z), lambda i: (0, 0)),
            pl.BlockSpec(memory_space=pl.ANY),
        ],
        out_specs=pl.BlockSpec((L_BLK * L, C_Z),
                               lambda i: (i + L_SC // L_BLK, 0)),
        out_shape=jax.ShapeDtypeStruct((L * L, C_Z), jnp.float32),
        input_output_aliases={7: 0},
    )(ptok_w, ptok_blk, ptab, wl_t, wr_t, b2, rel_pad, pair_low)

    msa_flat = pl.pallas_call(
        _msa_body,
        grid=(K * L // TOK_BLK,),
        in_specs=[
            pl.BlockSpec((1, 1, TOK_BLK), lambda i: (i, 0, 0)),
            pl.BlockSpec((VPAD, C_M), lambda i: (0, 0)),
        ],
        out_specs=pl.BlockSpec((TOK_BLK, C_M), lambda i: (i, 0)),
        out_shape=jax.ShapeDtypeStruct((K * L, C_M), jnp.float32),
    )(tok_flat, msa_pad)

    return (msa_flat.reshape(B, K, L, C_M), pair.reshape(B, L, L, C_Z))


# speedup vs baseline: 1.0717x; 1.0717x over previous
import functools

import jax
import jax.numpy as jnp
from jax import lax
from jax.experimental import pallas as pl
from jax.experimental.pallas import tpu as pltpu
from jax.experimental.pallas import tpu_sc as plsc

B, K, L = 1, 512, 256
VOCAB, C_M, C_Z = 5, 256, 128
D_EMB = C_Z // 2
MAXREL = 32
VPAD = 8
RELPAD = 72
TOK_BLK = 8192
E_ROWS = 512
NW = 32
L_PER_W = L // NW


def _msa_body(tok_ref, tab_ref, out_ref):
    t = jnp.broadcast_to(tok_ref[...].reshape(1, TOK_BLK), (VOCAB, TOK_BLK))
    vrow = jax.lax.broadcasted_iota(jnp.int32, (VOCAB, TOK_BLK), 0)
    onehot_t = (t == vrow).astype(jnp.float32)
    out_ref[...] = jax.lax.dot_general(
        onehot_t, tab_ref[...], (((0,), (0,)), ((), ())),
        preferred_element_type=jnp.float32)


def _pre_body(ptok_ref, ptab_ref, wl_ref, wr_ref, b_ref,
              lp_ref, rpb_ref):
    t = jnp.broadcast_to(ptok_ref[...].reshape(1, L), (VPAD, L))
    vrow = jax.lax.broadcasted_iota(jnp.int32, (VPAD, L), 0)
    onehot_t = (t == vrow).astype(jnp.float32)
    seq_emb = jax.lax.dot_general(
        onehot_t, ptab_ref[...], (((0,), (0,)), ((), ())),
        preferred_element_type=jnp.float32)
    lp_ref[...] = jnp.dot(seq_emb, wl_ref[...],
                          preferred_element_type=jnp.float32)
    rpb_ref[...] = jnp.dot(seq_emb, wr_ref[...],
                           preferred_element_type=jnp.float32) + b_ref[...]


def _sc_pair_body(lp_hbm, rpb_hbm, rel_hbm, out_hbm,
                  lp_v, rpb_v, rel_v, out_v, sem0, sem1, sem_in):
    cid = lax.axis_index("c")
    sid = lax.axis_index("s")
    wid = sid * 2 + cid
    l0 = wid * L_PER_W
    pltpu.async_copy(lp_hbm.at[pl.ds(l0, L_PER_W)], lp_v, sem_in).wait()
    pltpu.async_copy(rpb_hbm.at[pl.ds(0, L)], rpb_v, sem_in).wait()
    pltpu.async_copy(rel_hbm.at[pl.ds(0, RELPAD)], rel_v, sem_in).wait()

    sems = (sem0, sem1)
    pending = [None, None]
    for j in range(L_PER_W):
        p = j % 2
        if pending[p] is not None:
            pending[p].wait()
        l = l0 + j
        lo = lax.max(0, l - MAXREL)
        hi = lax.min(L - 1, l + MAXREL)
        lpc = [lp_v[j, pl.ds(c * 16, 16)] for c in range(8)]
        base0 = [lpc[c] + rel_v[0, pl.ds(c * 16, 16)] for c in range(8)]
        base64 = [lpc[c] + rel_v[2 * MAXREL, pl.ds(c * 16, 16)]
                  for c in range(8)]

        @plsc.parallel_loop(0, lo, unroll=4)
        def _lo_loop(m):
            vals = [rpb_v[m, pl.ds(c * 16, 16)] for c in range(8)]
            for c in range(8):
                out_v[p, m, pl.ds(c * 16, 16)] = vals[c] + base0[c]

        @plsc.parallel_loop(lo, hi + 1, unroll=2)
        def _band_loop(m):
            srow = m - l + MAXREL
            a = [rpb_v[m, pl.ds(c * 16, 16)] for c in range(8)]
            b = [rel_v[srow, pl.ds(c * 16, 16)] for c in range(8)]
            for c in range(8):
                out_v[p, m, pl.ds(c * 16, 16)] = a[c] + b[c] + lpc[c]

        @plsc.parallel_loop(hi + 1, L, unroll=4)
        def _hi_loop(m):
            vals = [rpb_v[m, pl.ds(c * 16, 16)] for c in range(8)]
            for c in range(8):
                out_v[p, m, pl.ds(c * 16, 16)] = vals[c] + base64[c]
        cp = pltpu.async_copy(out_v.at[p],
                              out_hbm.at[pl.ds(l * L, L)], sems[p])
        pending[p] = cp
    pending[0].wait()
    pending[1].wait()


_sc_pair = functools.partial(
    pl.kernel,
    out_type=jax.ShapeDtypeStruct((L * L, C_Z), jnp.float32),
    mesh=plsc.VectorSubcoreMesh(core_axis_name="c", subcore_axis_name="s"),
    scratch_types=[
        pltpu.VMEM((L_PER_W, C_Z), jnp.float32),
        pltpu.VMEM((L, C_Z), jnp.float32),
        pltpu.VMEM((RELPAD, C_Z), jnp.float32),
        pltpu.VMEM((2, L, C_Z), jnp.float32),
        pltpu.SemaphoreType.DMA,
        pltpu.SemaphoreType.DMA,
        pltpu.SemaphoreType.DMA,
    ],
)(_sc_pair_body)


@jax.jit
def kernel(tokens, pair_tokens, msa_table, pair_table, proj_W, proj_b,
           relpos_table):
    tok_flat = tokens.reshape(K * L // TOK_BLK, 1, TOK_BLK).astype(jnp.int32)
    ptok_w = pair_tokens.reshape(1, 1, L).astype(jnp.int32)
    ptab = jnp.zeros((VPAD, D_EMB), jnp.float32).at[:VOCAB].set(pair_table)
    wl_t = proj_W[:, :D_EMB].T
    wr_t = proj_W[:, D_EMB:].T
    b2 = proj_b.reshape(1, C_Z)
    rel_pad = jnp.zeros((RELPAD, C_Z), jnp.float32).at[:2 * MAXREL + 1].set(
        relpos_table)

    lp, rpb = pl.pallas_call(
        _pre_body,
        grid=(1,),
        in_specs=[
            pl.BlockSpec((1, 1, L), lambda i: (0, 0, 0)),
            pl.BlockSpec((VPAD, D_EMB), lambda i: (0, 0)),
            pl.BlockSpec((D_EMB, C_Z), lambda i: (0, 0)),
            pl.BlockSpec((D_EMB, C_Z), lambda i: (0, 0)),
            pl.BlockSpec((1, C_Z), lambda i: (0, 0)),
        ],
        out_specs=[
            pl.BlockSpec((L, C_Z), lambda i: (0, 0)),
            pl.BlockSpec((L, C_Z), lambda i: (0, 0)),
        ],
        out_shape=[
            jax.ShapeDtypeStruct((L, C_Z), jnp.float32),
            jax.ShapeDtypeStruct((L, C_Z), jnp.float32),
        ],
    )(ptok_w, ptab, wl_t, wr_t, b2)

    pair = _sc_pair(lp, rpb, rel_pad)

    msa_flat = pl.pallas_call(
        _msa_body,
        grid=(K * L // TOK_BLK,),
        in_specs=[
            pl.BlockSpec((1, 1, TOK_BLK), lambda i: (i, 0, 0)),
            pl.BlockSpec((VOCAB, C_M), lambda i: (0, 0)),
        ],
        out_specs=pl.BlockSpec((TOK_BLK, C_M), lambda i: (i, 0)),
        out_shape=jax.ShapeDtypeStruct((K * L, C_M), jnp.float32),
    )(tok_flat, msa_table)

    return (msa_flat.reshape(B, K, L, C_M), pair.reshape(B, L, L, C_Z))
